# baseline (speedup 1.0000x reference)
import jax
import jax.numpy as jnp
from jax import lax
from jax.experimental import pallas as pl
from jax.experimental.pallas import tpu as pltpu
from jax.experimental.pallas import tpu_sc as plsc

NC = 2
NS = 16
NW = NC * NS
PAIRS = 16384
D = 32
PAIRS_PER_W = PAIRS // NW
ROWS_PER_W = 2 * PAIRS_PER_W
NCHUNK = 8
CHUNK = ROWS_PER_W // NCHUNK
GROUPS = PAIRS_PER_W // 16


def _ip_body(emb_hbm, idx_hbm, out_hbm, idx_v, rows_v, out_v, sem):
    wid = lax.axis_index("s") * NC + lax.axis_index("c")
    pltpu.sync_copy(idx_hbm.at[wid], idx_v)
    copies = []
    for j in range(NCHUNK):
        copies.append(
            pltpu.async_copy(
                emb_hbm.at[idx_v.at[j]],
                rows_v.at[pl.ds(j * CHUNK, CHUNK)],
                sem,
            )
        )
    for c in copies:
        c.wait()

    even = lax.iota(jnp.int32, 16) * 2

    def group(g, carry):
        row_s = even + g * 32
        row_o = row_s + 1
        acc = jnp.zeros((16,), jnp.float32)
        for d in range(D):
            col = jnp.full((16,), d, jnp.int32)
            s_v = plsc.load_gather(rows_v, [row_s, col])
            o_v = plsc.load_gather(rows_v, [row_o, col])
            acc = acc + s_v * o_v
        out_v[pl.ds(g * 16, 16)] = 1.0 / (1.0 + jnp.exp(-acc))
        return carry

    lax.fori_loop(0, GROUPS, group, 0)
    pltpu.sync_copy(out_v, out_hbm.at[pl.ds(wid * PAIRS_PER_W, PAIRS_PER_W)])


@jax.jit
def _ip(emb, idx):
    mesh = plsc.VectorSubcoreMesh(core_axis_name="c", subcore_axis_name="s")
    return pl.kernel(
        _ip_body,
        mesh=mesh,
        compiler_params=pltpu.CompilerParams(
            needs_layout_passes=False, use_tc_tiling_on_sc=False
        ),
        out_type=jax.ShapeDtypeStruct((PAIRS,), jnp.float32),
        scratch_types=[
            pltpu.VMEM((NCHUNK, CHUNK), jnp.int32),
            pltpu.VMEM((ROWS_PER_W, D), jnp.float32),
            pltpu.VMEM((PAIRS_PER_W,), jnp.float32),
            pltpu.SemaphoreType.DMA,
        ],
    )(emb, idx)


def kernel(emb, batch_ind):
    idx = batch_ind.astype(jnp.int32).reshape(NW, NCHUNK, CHUNK)
    return _ip(emb, idx)
